# baseline (speedup 1.0000x reference)
import functools

import jax
import jax.numpy as jnp
from jax import lax
from jax.experimental import pallas as pl
from jax.experimental.pallas import tpu as pltpu
from jax.experimental.pallas import tpu_sc as plsc

B, T, IN, H = 64, 128, 1024, 4096
SRC = H + IN
BLK = 8192
L = 16


def _rnn_body(nblk, x_hbm, stream_hbm, bias_hbm, out_hbm,
              src0, src1, acc0, acc1, biasv, blkb, sem0, sem1):
    c = lax.axis_index("c")
    s = lax.axis_index("s")
    wid = s * 2 + c
    b0 = wid * 2
    b1 = b0 + 1
    sems = (sem0, sem1)

    pltpu.sync_copy(bias_hbm, biasv)

    @plsc.parallel_loop(0, H // L, unroll=4)
    def zinit(i):
        z = jnp.zeros((L,), jnp.float32)
        src0[pl.ds(i * L, L)] = z
        src1[pl.ds(i * L, L)] = z

    def start_blk(bi, slot):
        pltpu.async_copy(stream_hbm.at[bi], blkb.at[slot], sems[slot])

    def wait_blk(bi, slot):
        pltpu.make_async_copy(stream_hbm.at[bi], blkb.at[slot],
                              sems[slot]).wait()

    def step(t, carry):
        start_blk(0, 0)
        start_blk(1, 1)
        pltpu.sync_copy(x_hbm.at[b0, t], src0.at[pl.ds(H, IN)])
        pltpu.sync_copy(x_hbm.at[b1, t], src1.at[pl.ds(H, IN)])

        @plsc.parallel_loop(0, H // L, unroll=4)
        def binit(i):
            bv = biasv[pl.ds(i * L, L)]
            acc0[pl.ds(i * L, L)] = bv
            acc1[pl.ds(i * L, L)] = bv

        def pair(g, cc):
            for slot in range(2):
                bi = g * 2 + slot
                wait_blk(bi, slot)

                @plsc.parallel_loop(0, BLK // L, unroll=8)
                def inner(i):
                    pw = blkb[slot, 0, pl.ds(i * L, L)]
                    vv = plsc.bitcast(blkb[slot, 1, pl.ds(i * L, L)],
                                      jnp.float32)
                    colsv = jnp.bitwise_and(pw, 8191)
                    rowsv = jnp.right_shift(pw, 13)
                    g0 = plsc.load_gather(src0, [colsv])
                    plsc.addupdate_scatter(acc0, [rowsv], g0 * vv)
                    g1 = plsc.load_gather(src1, [colsv])
                    plsc.addupdate_scatter(acc1, [rowsv], g1 * vv)

                @pl.when(bi + 2 < nblk)
                def _():
                    start_blk(bi + 2, slot)
            return cc

        lax.fori_loop(0, nblk // 2, pair, 0)

        @plsc.parallel_loop(0, H // L, unroll=4)
        def finish(i):
            sl = pl.ds(i * L, L)
            z0 = acc0[sl]
            e0 = jnp.exp(z0 + z0)
            src0[sl] = 1.0 - 2.0 / (e0 + 1.0)
            z1 = acc1[sl]
            e1 = jnp.exp(z1 + z1)
            src1[sl] = 1.0 - 2.0 / (e1 + 1.0)

        pltpu.sync_copy(src0.at[pl.ds(0, H)], out_hbm.at[b0, t])
        pltpu.sync_copy(src1.at[pl.ds(0, H)], out_hbm.at[b1, t])
        return carry

    lax.fori_loop(0, T, step, 0)


def kernel(x, idx_hh, values_hh, idx_ih, values_ih, bias_hh):
    packed_hh = idx_hh[0] * 8192 + idx_hh[1]
    packed_ih = idx_ih[0] * 8192 + (idx_ih[1] + H)
    packed = jnp.concatenate([packed_hh, packed_ih])
    vals = jnp.concatenate([values_hh, values_ih])
    nnz = packed.shape[0]

    col = packed & 8191
    row = packed >> 13
    ccls = col % L
    dcls = (row - col) % L
    order = jnp.argsort(ccls * L + dcls)
    m = (nnz + L - 1) // L
    spad = m * L - nnz
    packed = jnp.pad(packed[order], (0, spad))
    vals = jnp.pad(vals[order], (0, spad))
    packed = packed.reshape(L, m).T.reshape(-1)
    vals = vals.reshape(L, m).T.reshape(-1)
    nnz = m * L

    nblk = (nnz + BLK - 1) // BLK
    if nblk % 2:
        nblk += 1
    pad = nblk * BLK - nnz
    packed = jnp.pad(packed, (0, pad))
    vals = jnp.pad(vals, (0, pad))
    valbits = jax.lax.bitcast_convert_type(vals, jnp.int32)
    stream = jnp.stack([packed.reshape(nblk, BLK),
                        valbits.reshape(nblk, BLK)], axis=1)
    bias = bias_hh[:, 0]

    mesh = plsc.VectorSubcoreMesh(core_axis_name="c", subcore_axis_name="s")
    run = pl.kernel(
        functools.partial(_rnn_body, nblk),
        out_type=jax.ShapeDtypeStruct((B, T, H), jnp.float32),
        mesh=mesh,
        compiler_params=pltpu.CompilerParams(needs_layout_passes=False),
        scratch_types=[
            pltpu.VMEM((SRC,), jnp.float32),
            pltpu.VMEM((SRC,), jnp.float32),
            pltpu.VMEM((H,), jnp.float32),
            pltpu.VMEM((H,), jnp.float32),
            pltpu.VMEM((H,), jnp.float32),
            pltpu.VMEM((2, 2, BLK), jnp.int32),
            pltpu.SemaphoreType.DMA,
            pltpu.SemaphoreType.DMA,
        ],
    )
    return run(x, stream, bias)


# speedup vs baseline: 1.1695x; 1.1695x over previous
import functools

import jax
import jax.numpy as jnp
from jax import lax
from jax.experimental import pallas as pl
from jax.experimental.pallas import tpu as pltpu
from jax.experimental.pallas import tpu_sc as plsc

B, T, IN, H = 64, 128, 1024, 4096
SRC = H + IN
BLK = 8192
L = 16


def _rnn_body(nblk, x_hbm, stream_hbm, bias_hbm, out_hbm,
              src0, src1, acc0, acc1, biasv, blkb, sem0, sem1):
    c = lax.axis_index("c")
    s = lax.axis_index("s")
    wid = s * 2 + c
    b0 = wid * 2
    b1 = b0 + 1
    sems = (sem0, sem1)

    pltpu.sync_copy(bias_hbm, biasv)

    @plsc.parallel_loop(0, H // L, unroll=4)
    def zinit(i):
        z = jnp.zeros((L,), jnp.float32)
        src0[pl.ds(i * L, L)] = z
        src1[pl.ds(i * L, L)] = z

    def start_blk(bi, slot):
        pltpu.async_copy(stream_hbm.at[bi], blkb.at[slot], sems[slot])

    def wait_blk(bi, slot):
        pltpu.make_async_copy(stream_hbm.at[bi], blkb.at[slot],
                              sems[slot]).wait()

    def step(t, carry):
        start_blk(0, 0)
        start_blk(1, 1)
        pltpu.sync_copy(x_hbm.at[b0, t], src0.at[pl.ds(H, IN)])
        pltpu.sync_copy(x_hbm.at[b1, t], src1.at[pl.ds(H, IN)])

        @plsc.parallel_loop(0, H // L, unroll=4)
        def binit(i):
            bv = biasv[pl.ds(i * L, L)]
            acc0[pl.ds(i * L, L)] = bv
            acc1[pl.ds(i * L, L)] = bv

        def pair(g, cc):
            for slot in range(2):
                bi = g * 2 + slot
                wait_blk(bi, slot)

                @plsc.parallel_loop(0, BLK // L, unroll=8)
                def inner(i):
                    pw = blkb[slot, 0, pl.ds(i * L, L)]
                    vv = plsc.bitcast(blkb[slot, 1, pl.ds(i * L, L)],
                                      jnp.float32)
                    colsv = jnp.bitwise_and(pw, 8191)
                    rowsv = jnp.right_shift(pw, 13)
                    g0 = plsc.load_gather(src0, [colsv])
                    plsc.addupdate_scatter(acc0, [rowsv], g0 * vv)
                    g1 = plsc.load_gather(src1, [colsv])
                    plsc.addupdate_scatter(acc1, [rowsv], g1 * vv)

                @pl.when(bi + 2 < nblk)
                def _():
                    start_blk(bi + 2, slot)
            return cc

        lax.fori_loop(0, nblk // 2, pair, 0)

        @plsc.parallel_loop(0, H // L, unroll=4)
        def finish(i):
            sl = pl.ds(i * L, L)
            z0 = acc0[sl]
            e0 = jnp.exp(z0 + z0)
            src0[sl] = 1.0 - 2.0 / (e0 + 1.0)
            z1 = acc1[sl]
            e1 = jnp.exp(z1 + z1)
            src1[sl] = 1.0 - 2.0 / (e1 + 1.0)

        pltpu.sync_copy(src0.at[pl.ds(0, H)], out_hbm.at[b0, t])
        pltpu.sync_copy(src1.at[pl.ds(0, H)], out_hbm.at[b1, t])
        return carry

    lax.fori_loop(0, T, step, 0)


def kernel(x, idx_hh, values_hh, idx_ih, values_ih, bias_hh):
    packed_hh = idx_hh[0] * 8192 + idx_hh[1]
    packed_ih = idx_ih[0] * 8192 + (idx_ih[1] + H)
    packed = jnp.concatenate([packed_hh, packed_ih])
    vals = jnp.concatenate([values_hh, values_ih])
    nnz = packed.shape[0]

    nblk = (nnz + BLK - 1) // BLK
    if nblk % 2:
        nblk += 1
    pad = nblk * BLK - nnz
    packed = jnp.pad(packed, (0, pad))
    vals = jnp.pad(vals, (0, pad))
    valbits = jax.lax.bitcast_convert_type(vals, jnp.int32)
    stream = jnp.stack([packed.reshape(nblk, BLK),
                        valbits.reshape(nblk, BLK)], axis=1)
    bias = bias_hh[:, 0]

    mesh = plsc.VectorSubcoreMesh(core_axis_name="c", subcore_axis_name="s")
    run = pl.kernel(
        functools.partial(_rnn_body, nblk),
        out_type=jax.ShapeDtypeStruct((B, T, H), jnp.float32),
        mesh=mesh,
        compiler_params=pltpu.CompilerParams(needs_layout_passes=False),
        scratch_types=[
            pltpu.VMEM((SRC,), jnp.float32),
            pltpu.VMEM((SRC,), jnp.float32),
            pltpu.VMEM((H,), jnp.float32),
            pltpu.VMEM((H,), jnp.float32),
            pltpu.VMEM((H,), jnp.float32),
            pltpu.VMEM((2, 2, BLK), jnp.int32),
            pltpu.SemaphoreType.DMA,
            pltpu.SemaphoreType.DMA,
        ],
    )
    return run(x, stream, bias)


# speedup vs baseline: 1.3356x; 1.1421x over previous
import functools

import jax
import jax.numpy as jnp
from jax import lax
from jax.experimental import pallas as pl
from jax.experimental.pallas import tpu as pltpu
from jax.experimental.pallas import tpu_sc as plsc

B, T, IN, H = 64, 128, 1024, 4096
SRC = H + IN
BLK = 8192
L = 16


def _rnn_body(nblk, xp_hbm, stream_hbm, bias_hbm, out_hbm,
              srcp, acc0, acc1, biasv, blkb, sem0, sem1):
    c = lax.axis_index("c")
    s = lax.axis_index("s")
    wid = s * 2 + c
    b0 = wid * 2
    b1 = b0 + 1
    sems = (sem0, sem1)

    pltpu.sync_copy(bias_hbm, biasv)

    @plsc.parallel_loop(0, H // L, unroll=4)
    def zinit(i):
        srcp[pl.ds(i * L, L)] = jnp.zeros((L,), jnp.int32)

    def start_blk(bi, slot):
        pltpu.async_copy(stream_hbm.at[bi], blkb.at[slot], sems[slot])

    def wait_blk(bi, slot):
        pltpu.make_async_copy(stream_hbm.at[bi], blkb.at[slot],
                              sems[slot]).wait()

    def step(t, carry):
        start_blk(0, 0)
        start_blk(1, 1)
        pltpu.sync_copy(xp_hbm.at[wid, t], srcp.at[pl.ds(H, IN)])

        @plsc.parallel_loop(0, H // L, unroll=4)
        def binit(i):
            bv = biasv[pl.ds(i * L, L)]
            acc0[pl.ds(i * L, L)] = bv
            acc1[pl.ds(i * L, L)] = bv

        def pair(g, cc):
            for slot in range(2):
                bi = g * 2 + slot
                wait_blk(bi, slot)

                @plsc.parallel_loop(0, BLK // L, unroll=8)
                def inner(i):
                    pw = blkb[slot, 0, pl.ds(i * L, L)]
                    vv = plsc.bitcast(blkb[slot, 1, pl.ds(i * L, L)],
                                      jnp.float32)
                    colsv = jnp.bitwise_and(pw, 8191)
                    rowsv = jnp.right_shift(pw, 13)
                    gp = plsc.load_gather(srcp, [colsv])
                    g0, g1 = plsc.unpack(
                        plsc.bitcast(gp, jnp.bfloat16),
                        format=plsc.PackFormat.INTERLEAVED)
                    plsc.addupdate_scatter(acc0, [rowsv], g0 * vv)
                    plsc.addupdate_scatter(acc1, [rowsv], g1 * vv)

                @pl.when(bi + 2 < nblk)
                def _():
                    start_blk(bi + 2, slot)
            return cc

        lax.fori_loop(0, nblk // 2, pair, 0)

        @plsc.parallel_loop(0, H // L, unroll=4)
        def finish(i):
            sl = pl.ds(i * L, L)
            z0 = acc0[sl]
            e0 = jnp.exp(z0 + z0)
            h0 = 1.0 - 2.0 / (e0 + 1.0)
            z1 = acc1[sl]
            e1 = jnp.exp(z1 + z1)
            h1 = 1.0 - 2.0 / (e1 + 1.0)
            acc0[sl] = h0
            acc1[sl] = h1
            hp = plsc.pack(h0, h1, format=plsc.PackFormat.INTERLEAVED)
            srcp[sl] = plsc.bitcast(hp, jnp.int32)

        pltpu.sync_copy(acc0, out_hbm.at[b0, t])
        pltpu.sync_copy(acc1, out_hbm.at[b1, t])
        return carry

    lax.fori_loop(0, T, step, 0)


def kernel(x, idx_hh, values_hh, idx_ih, values_ih, bias_hh):
    packed_hh = idx_hh[0] * 8192 + idx_hh[1]
    packed_ih = idx_ih[0] * 8192 + (idx_ih[1] + H)
    packed = jnp.concatenate([packed_hh, packed_ih])
    vals = jnp.concatenate([values_hh, values_ih])
    nnz = packed.shape[0]

    nblk = (nnz + BLK - 1) // BLK
    if nblk % 2:
        nblk += 1
    pad = nblk * BLK - nnz
    packed = jnp.pad(packed, (0, pad))
    vals = jnp.pad(vals, (0, pad))
    valbits = jax.lax.bitcast_convert_type(vals, jnp.int32)
    stream = jnp.stack([packed.reshape(nblk, BLK),
                        valbits.reshape(nblk, BLK)], axis=1)
    bias = bias_hh[:, 0]

    xb = x.astype(jnp.bfloat16)
    xu = jax.lax.bitcast_convert_type(xb, jnp.uint16).astype(jnp.uint32)
    xp = jax.lax.bitcast_convert_type(
        xu[0::2] | (xu[1::2] << 16), jnp.int32)

    mesh = plsc.VectorSubcoreMesh(core_axis_name="c", subcore_axis_name="s")
    run = pl.kernel(
        functools.partial(_rnn_body, nblk),
        out_type=jax.ShapeDtypeStruct((B, T, H), jnp.float32),
        mesh=mesh,
        compiler_params=pltpu.CompilerParams(needs_layout_passes=False),
        scratch_types=[
            pltpu.VMEM((SRC,), jnp.int32),
            pltpu.VMEM((H,), jnp.float32),
            pltpu.VMEM((H,), jnp.float32),
            pltpu.VMEM((H,), jnp.float32),
            pltpu.VMEM((2, 2, BLK), jnp.int32),
            pltpu.SemaphoreType.DMA,
            pltpu.SemaphoreType.DMA,
        ],
    )
    return run(xp, stream, bias)
